# baseline (speedup 1.0000x reference)
import functools

import jax
import jax.numpy as jnp
from jax import lax
from jax.experimental import pallas as pl
from jax.experimental.pallas import tpu as pltpu
from jax.experimental.pallas import tpu_sc as plsc

NC, NS, LANES = 2, 16, 16
NW = NC * NS
D = 64
SCALE = 8.0
CW = 512


@functools.lru_cache(maxsize=None)
def _build(B: int):
    assert B % (NW * CW) == 0, B
    bpw = B // NW
    nwin = bpw // CW
    mesh = plsc.VectorSubcoreMesh(core_axis_name="c", subcore_axis_name="s")

    @functools.partial(
        pl.kernel,
        out_type=jax.ShapeDtypeStruct((B, D), jnp.float32),
        mesh=mesh,
        scratch_types=[
            pltpu.VMEM((bpw,), jnp.int32),
            pltpu.VMEM((CW, D), jnp.float32),
            pltpu.SemaphoreType.DMA,
        ],
        compiler_params=pltpu.CompilerParams(use_tc_tiling_on_sc=False),
    )
    def emb_kernel(x_hbm, emb_hbm, out_hbm, idx_all, rows, gsem):
        wid = lax.axis_index("s") * NC + lax.axis_index("c")
        base = wid * bpw
        pltpu.sync_copy(x_hbm.at[pl.ds(base, bpw)], idx_all)

        def window(g, carry):
            woff = g * CW
            for j in range(CW // LANES):
                iv = idx_all[pl.ds(woff + j * LANES, LANES)]
                pltpu.async_copy(emb_hbm.at[iv],
                                 rows.at[pl.ds(j * LANES, LANES)], gsem)
            pltpu.make_async_copy(out_hbm.at[pl.ds(base + woff, CW)], rows,
                                  gsem).wait()

            @plsc.parallel_loop(0, CW, step=1, unroll=8)
            def _scale(i):
                for k in range(D // LANES):
                    sl = pl.ds(k * LANES, LANES)
                    rows[i, sl] = rows[i, sl] * SCALE

            pltpu.sync_copy(rows, out_hbm.at[pl.ds(base + woff, CW)])
            return carry

        lax.fori_loop(0, nwin, window, 0)

    return emb_kernel


def kernel(x, emb):
    s0, s1 = x.shape
    xf = x.reshape(-1).astype(jnp.int32)
    out = _build(s0 * s1)(xf, emb)
    return out.reshape(s0, s1, D)
